# baseline (speedup 1.0000x reference)
import functools

import jax
import jax.numpy as jnp
from jax import lax
from jax.experimental import pallas as pl
from jax.experimental.pallas import tpu as pltpu
from jax.experimental.pallas import tpu_sc as plsc

N = 10000
E = 320000
D = 200
B = 1024
ALPHA = 0.01

FA = 111
DPA = 112
DT = 128
FB = D - FA
DP = 208

EPS = E // 16
CH = 80
NCH = EPS // CH
ROWS_PT = N // 16
BR = 1000
GRID = N // BR

_SC_PARAMS = pltpu.CompilerParams(
    use_tc_tiling_on_sc=False, needs_layout_passes=False)


def _split_tables(h):
    ones = jnp.ones((BR, 1), jnp.float32)
    ha = jnp.concatenate(
        [h[:, :FA], ones, jnp.zeros((BR, DT - FA - 1), jnp.float32)], axis=1)
    hb = jnp.concatenate(
        [h[:, FA:], ones, jnp.zeros((BR, DT - FB - 1), jnp.float32)], axis=1)
    hc = jnp.concatenate(
        [ha.reshape(1, BR, DT), hb.reshape(1, BR, DT)], axis=0)
    hc = hc.reshape(2, BR, DT // 32, 2, 16).transpose(0, 1, 2, 4, 3)
    return hc.reshape(2, BR, DT).astype(jnp.bfloat16)


def _prep_block(x_ref, wl_ref, av_ref, w_ref, b_ref,
                hc_ref, ad_ref, as_ref, sg_ref):
    x = x_ref[...]
    h = jnp.dot(x, wl_ref[...], preferred_element_type=jnp.float32)
    hc_ref[...] = _split_tables(h)
    coef = lax.dot_general(h, av_ref[...], (((1,), (1,)), ((), ())),
                           preferred_element_type=jnp.float32)
    ad_ref[...] = coef[:, 0].reshape(1, 1, BR)
    as_ref[...] = coef[:, 1].reshape(1, 1, BR)
    sg_ref[...] = jax.nn.sigmoid(
        jnp.dot(x, w_ref[...], preferred_element_type=jnp.float32) + b_ref[...])


def _tc_prep(x, wl, av, w, b):
    return pl.pallas_call(
        _prep_block,
        grid=(GRID,),
        in_specs=[
            pl.BlockSpec((BR, D), lambda i: (i, 0)),
            pl.BlockSpec((D, D), lambda i: (0, 0)),
            pl.BlockSpec((2, D), lambda i: (0, 0)),
            pl.BlockSpec((D, D), lambda i: (0, 0)),
            pl.BlockSpec((1, D), lambda i: (0, 0)),
        ],
        out_specs=[
            pl.BlockSpec((2, BR, DT), lambda i: (0, i, 0)),
            pl.BlockSpec((1, 1, BR), lambda i: (i, 0, 0)),
            pl.BlockSpec((1, 1, BR), lambda i: (i, 0, 0)),
            pl.BlockSpec((BR, D), lambda i: (i, 0)),
        ],
        out_shape=[
            jax.ShapeDtypeStruct((2, N, DT), jnp.bfloat16),
            jax.ShapeDtypeStruct((GRID, 1, BR), jnp.float32),
            jax.ShapeDtypeStruct((GRID, 1, BR), jnp.float32),
            jax.ShapeDtypeStruct((N, D), jnp.float32),
        ],
    )(x, wl, av, w, b)


def _gate_combine(p_ref, sg_ref, x_ref):
    p0 = p_ref[0]
    p1 = p_ref[1]
    s = p0[:, FA:FA + 1]
    agg = jnp.concatenate([p0[:, :FA], p1[:, :FB]], axis=1)
    out = jax.nn.sigmoid(agg / (s + 1e-16))
    sg = sg_ref[...]
    return sg * out + (1.0 - sg) * x_ref[...]


def _combine_prep_block(p_ref, sg_ref, x_ref, wl_ref, av_ref, w_ref, b_ref,
                        x1_ref, hc_ref, ad_ref, as_ref, sg1_ref):
    x1 = _gate_combine(p_ref, sg_ref, x_ref)
    x1_ref[...] = x1
    h = jnp.dot(x1, wl_ref[...], preferred_element_type=jnp.float32)
    hc_ref[...] = _split_tables(h)
    coef = lax.dot_general(h, av_ref[...], (((1,), (1,)), ((), ())),
                           preferred_element_type=jnp.float32)
    ad_ref[...] = coef[:, 0].reshape(1, 1, BR)
    as_ref[...] = coef[:, 1].reshape(1, 1, BR)
    sg1_ref[...] = jax.nn.sigmoid(
        jnp.dot(x1, w_ref[...], preferred_element_type=jnp.float32) + b_ref[...])


def _tc_combine_prep(parts, sg, x, wl, av, w, b):
    return pl.pallas_call(
        _combine_prep_block,
        grid=(GRID,),
        in_specs=[
            pl.BlockSpec((2, BR, DPA), lambda i: (0, i, 0)),
            pl.BlockSpec((BR, D), lambda i: (i, 0)),
            pl.BlockSpec((BR, D), lambda i: (i, 0)),
            pl.BlockSpec((D, D), lambda i: (0, 0)),
            pl.BlockSpec((2, D), lambda i: (0, 0)),
            pl.BlockSpec((D, D), lambda i: (0, 0)),
            pl.BlockSpec((1, D), lambda i: (0, 0)),
        ],
        out_specs=[
            pl.BlockSpec((BR, D), lambda i: (i, 0)),
            pl.BlockSpec((2, BR, DT), lambda i: (0, i, 0)),
            pl.BlockSpec((1, 1, BR), lambda i: (i, 0, 0)),
            pl.BlockSpec((1, 1, BR), lambda i: (i, 0, 0)),
            pl.BlockSpec((BR, D), lambda i: (i, 0)),
        ],
        out_shape=[
            jax.ShapeDtypeStruct((N, D), jnp.float32),
            jax.ShapeDtypeStruct((2, N, DT), jnp.bfloat16),
            jax.ShapeDtypeStruct((GRID, 1, BR), jnp.float32),
            jax.ShapeDtypeStruct((GRID, 1, BR), jnp.float32),
            jax.ShapeDtypeStruct((N, D), jnp.float32),
        ],
    )(parts, sg, x, wl, av, w, b)


def _combine_final_block(p_ref, sg_ref, x_ref, x2p_ref):
    x2 = _gate_combine(p_ref, sg_ref, x_ref)
    x2p_ref[...] = jnp.concatenate(
        [x2, jnp.zeros((BR, DP - D), jnp.float32)], axis=1)


def _tc_combine_final(parts, sg, x):
    return pl.pallas_call(
        _combine_final_block,
        grid=(GRID,),
        in_specs=[
            pl.BlockSpec((2, BR, DPA), lambda i: (0, i, 0)),
            pl.BlockSpec((BR, D), lambda i: (i, 0)),
            pl.BlockSpec((BR, D), lambda i: (i, 0)),
        ],
        out_specs=[pl.BlockSpec((BR, DP), lambda i: (i, 0))],
        out_shape=[jax.ShapeDtypeStruct((N, DP), jnp.float32)],
    )(parts, sg, x)


def _sc_edges(src, dst, adst, asrc, hcat, zeros):
    mesh = plsc.VectorSubcoreMesh(core_axis_name="c", subcore_axis_name="s")
    def nbuf_scratch():
        return [
            pltpu.VMEM((CH,), jnp.int32),
            pltpu.VMEM((CH,), jnp.int32),
            pltpu.VMEM((CH,), jnp.int32),
            pltpu.VMEM((CH,), jnp.int32),
            pltpu.VMEM((CH,), jnp.float32),
            pltpu.VMEM((CH, DT), jnp.bfloat16),
            pltpu.VMEM((CH, DPA), jnp.float32),
            pltpu.SemaphoreType.DMA,
            pltpu.SemaphoreType.DMA,
        ]

    @functools.partial(
        pl.kernel,
        mesh=mesh,
        out_type=jax.ShapeDtypeStruct((2, N, DPA), jnp.float32),
        compiler_params=_SC_PARAMS,
        scratch_types=[
            pltpu.VMEM((N,), jnp.float32),
            pltpu.VMEM((N,), jnp.float32),
        ] + nbuf_scratch() + nbuf_scratch() + [
            pltpu.SemaphoreType.DMA,
            pltpu.VMEM_SHARED((N, DPA), jnp.float32),
        ],
    )
    def body(src_h, dst_h, adst_h, asrc_h, hcat_h, zeros_h, parts_h,
             adst_v, asrc_v, *bufs_and_acc):
        bufs = bufs_and_acc[:18]
        ssem = bufs_and_acc[18]
        acc = bufs_and_acc[19]
        cid = lax.axis_index("c")
        sid = lax.axis_index("s")
        pltpu.sync_copy(adst_h, adst_v)
        pltpu.sync_copy(asrc_h, asrc_v)
        pltpu.sync_copy(zeros_h, acc.at[pl.ds(sid * ROWS_PT, ROWS_PT)])
        plsc.subcore_barrier()
        rowoff = cid * N
        b0 = bufs[:9]
        b1 = bufs[9:]

        def chunk_off(c):
            return pl.multiple_of(
                jnp.minimum(sid * EPS + c * CH, E - CH), 8)

        def p1(c, buf):
            srcv, dstv, _, _, _, _, _, isem, _ = buf
            off = chunk_off(c)
            pltpu.async_copy(src_h.at[pl.ds(off, CH)], srcv, isem)
            pltpu.async_copy(dst_h.at[pl.ds(off, CH)], dstv, isem)

        def p1_wait(c, buf):
            srcv, dstv, _, _, _, _, _, isem, _ = buf
            off = chunk_off(c)
            pltpu.make_async_copy(src_h.at[pl.ds(off, CH)], srcv, isem).wait()
            pltpu.make_async_copy(dst_h.at[pl.ds(off, CH)], dstv, isem).wait()

        def p2(c, buf):
            srcv, dstv, sdstv, gidx, exv, rows, srows, isem, gsem = buf
            p1_wait(c, buf)
            for j in range(CH // 16):
                sl = pl.ds(j * 16, 16)
                gidx[sl] = srcv[sl] + rowoff
            pltpu.async_copy(hcat_h.at[gidx], rows, gsem)
            for j in range(CH // 16):
                sl = pl.ds(j * 16, 16)
                sdstv[sl] = dstv[sl]
                e = (plsc.load_gather(adst_v, [dstv[sl]])
                     + plsc.load_gather(asrc_v, [srcv[sl]]))
                e = jnp.where(e > 0, e, ALPHA * e)
                exv[sl] = jnp.exp(e)

        def drain_scatter(buf):
            srows = buf[6]
            pltpu.make_async_copy(zeros_h.at[pl.ds(0, CH)], srows, ssem).wait()

        def p3(buf, prev_buf):
            srcv, dstv, sdstv, gidx, exv, rows, srows, isem, gsem = buf
            pltpu.make_async_copy(hcat_h.at[gidx], rows, gsem).wait()
            for j in range(CH // 16):
                ev = exv[pl.ds(j * 16, 16)]
                for l in range(16):
                    w = lax.reshape(lax.slice(ev, (l,), (l + 1,)), ())
                    r = j * 16 + l
                    for k in range(DT // 32):
                        va, vb = plsc.unpack(
                            rows[r, pl.ds(k * 32, 32)],
                            format=plsc.PackFormat.INTERLEAVED)
                        srows[r, pl.ds(k * 32, 16)] = va * w
                        if k * 32 + 32 <= DPA:
                            srows[r, pl.ds(k * 32 + 16, 16)] = vb * w
            drain_scatter(prev_buf)
            pltpu.async_copy(srows, acc.at[sdstv], ssem, add=True)

        for j in range(CH // 16):
            b1[2][pl.ds(j * 16, 16)] = jnp.zeros((16,), jnp.int32)
        pltpu.sync_copy(zeros_h.at[pl.ds(0, CH)], b1[6])
        pltpu.async_copy(b1[6], acc.at[b1[2]], ssem, add=True)
        p1(0, b0)
        p1(1, b1)
        p2(0, b0)
        p2(1, b1)
        p1(2, b0)
        p1(3, b1)

        def superstep(g, carry):
            c0 = 2 * g
            p3(b0, b1)
            p3(b1, b0)
            p2(c0 + 2, b0)
            p2(c0 + 3, b1)
            p1(c0 + 4, b0)
            p1(c0 + 5, b1)
            return carry

        lax.fori_loop(0, NCH // 2, superstep, 0)

        for c, buf in ((NCH + 2, b0), (NCH + 3, b1)):
            srcv, dstv, sdstv, gidx, exv, rows, srows, isem, gsem = buf
            p1_wait(c, buf)
            pltpu.make_async_copy(hcat_h.at[gidx], rows, gsem).wait()
        drain_scatter(b1)
        plsc.subcore_barrier()
        pltpu.sync_copy(acc.at[pl.ds(sid * ROWS_PT, ROWS_PT)],
                        parts_h.at[cid, pl.ds(sid * ROWS_PT, ROWS_PT)])

    return body(src, dst, adst, asrc, hcat, zeros)


def _sc_batch_gather(bh, bt, br, x2p, relp):
    mesh = plsc.VectorSubcoreMesh(core_axis_name="c", subcore_axis_name="s")
    bpw = B // 32

    @functools.partial(
        pl.kernel,
        mesh=mesh,
        out_type=(
            jax.ShapeDtypeStruct((B, DP), jnp.float32),
            jax.ShapeDtypeStruct((B, DP), jnp.float32),
            jax.ShapeDtypeStruct((B, DP), jnp.float32),
        ),
        compiler_params=_SC_PARAMS,
        scratch_types=[
            pltpu.VMEM((bpw,), jnp.int32),
            pltpu.VMEM((bpw, DP), jnp.float32),
            pltpu.SemaphoreType.DMA,
        ],
    )
    def body(bh_h, bt_h, br_h, x2p_h, relp_h, oh_h, ot_h, orr_h,
             idxv, rows, sem):
        cid = lax.axis_index("c")
        sid = lax.axis_index("s")
        base = pl.multiple_of((cid * 16 + sid) * bpw, 8)
        for idx_h, tab_h, out_h in ((bh_h, x2p_h, oh_h),
                                    (bt_h, x2p_h, ot_h),
                                    (br_h, relp_h, orr_h)):
            pltpu.sync_copy(idx_h.at[pl.ds(base, bpw)], idxv)
            pltpu.async_copy(tab_h.at[idxv], rows, sem).wait()
            pltpu.sync_copy(rows, out_h.at[pl.ds(base, bpw)])

    return body(bh, bt, br, x2p, relp)


def kernel(batch_h, batch_r, batch_t, edge_index, ent_embed, rel_embed,
           W, b, Wl0, a0, Wl1, a1):
    src = edge_index[0].astype(jnp.int32)
    dst = edge_index[1].astype(jnp.int32)
    av0 = a0.reshape(2, D)
    av1 = a1.reshape(2, D)
    zeros = jnp.zeros((ROWS_PT, DPA), jnp.float32)

    hc0, ad0, as0, sg0 = _tc_prep(ent_embed, Wl0, av0, W, b)
    parts0 = _sc_edges(src, dst, ad0.reshape(N), as0.reshape(N),
                       hc0.reshape(2 * N, DT), zeros)
    x1, hc1, ad1, as1, sg1 = _tc_combine_prep(parts0, sg0, ent_embed,
                                              Wl1, av1, W, b)
    parts1 = _sc_edges(src, dst, ad1.reshape(N), as1.reshape(N),
                       hc1.reshape(2 * N, DT), zeros)
    (x2p,) = _tc_combine_final(parts1, sg1, x1)

    relp = jnp.pad(rel_embed, ((0, 0), (0, DP - D)))
    oh, ot, orr = _sc_batch_gather(batch_h.astype(jnp.int32),
                                   batch_t.astype(jnp.int32),
                                   batch_r.astype(jnp.int32), x2p, relp)
    return oh[:, :D], orr[:, :D], ot[:, :D]


# speedup vs baseline: 1.7540x; 1.7540x over previous
import functools

import jax
import jax.numpy as jnp
from jax import lax
from jax.experimental import pallas as pl
from jax.experimental.pallas import tpu as pltpu
from jax.experimental.pallas import tpu_sc as plsc

N = 10000
E = 320000
D = 200
B = 1024
ALPHA = 0.01

FA = 111
DPA = 112
FB = D - FA
DP = 208

EPS = E // 16
CH = 80
NCH = EPS // CH
ROWS_PT = N // 16
BR = 2000
GRID = N // BR

_SC_PARAMS = pltpu.CompilerParams(
    use_tc_tiling_on_sc=False, needs_layout_passes=False)


def _split_tables(h):
    ones = jnp.ones((BR, 1), jnp.float32)
    ha = jnp.concatenate([h[:, :FA], ones], axis=1)
    hb = jnp.concatenate(
        [h[:, FA:], ones, jnp.zeros((BR, DPA - FB - 1), jnp.float32)], axis=1)
    return jnp.concatenate(
        [ha.reshape(1, BR, DPA), hb.reshape(1, BR, DPA)], axis=0)


def _prep_block(x_ref, wl_ref, av_ref, w_ref, b_ref,
                hc_ref, ad_ref, as_ref, sg_ref):
    x = x_ref[...]
    h = jnp.dot(x, wl_ref[...], preferred_element_type=jnp.float32)
    hc_ref[...] = _split_tables(h)
    coef = lax.dot_general(h, av_ref[...], (((1,), (1,)), ((), ())),
                           preferred_element_type=jnp.float32)
    ad_ref[...] = coef[:, 0].reshape(1, 1, BR)
    as_ref[...] = coef[:, 1].reshape(1, 1, BR)
    sg_ref[...] = jax.nn.sigmoid(
        jnp.dot(x, w_ref[...], preferred_element_type=jnp.float32) + b_ref[...])


def _tc_prep(x, wl, av, w, b):
    return pl.pallas_call(
        _prep_block,
        grid=(GRID,),
        in_specs=[
            pl.BlockSpec((BR, D), lambda i: (i, 0)),
            pl.BlockSpec((D, D), lambda i: (0, 0)),
            pl.BlockSpec((2, D), lambda i: (0, 0)),
            pl.BlockSpec((D, D), lambda i: (0, 0)),
            pl.BlockSpec((1, D), lambda i: (0, 0)),
        ],
        out_specs=[
            pl.BlockSpec((2, BR, DPA), lambda i: (0, i, 0)),
            pl.BlockSpec((1, 1, BR), lambda i: (i, 0, 0)),
            pl.BlockSpec((1, 1, BR), lambda i: (i, 0, 0)),
            pl.BlockSpec((BR, D), lambda i: (i, 0)),
        ],
        out_shape=[
            jax.ShapeDtypeStruct((2, N, DPA), jnp.float32),
            jax.ShapeDtypeStruct((GRID, 1, BR), jnp.float32),
            jax.ShapeDtypeStruct((GRID, 1, BR), jnp.float32),
            jax.ShapeDtypeStruct((N, D), jnp.float32),
        ],
    )(x, wl, av, w, b)


def _gate_combine(p_ref, sg_ref, x_ref):
    p0 = p_ref[0]
    p1 = p_ref[1]
    s = p0[:, FA:FA + 1]
    agg = jnp.concatenate([p0[:, :FA], p1[:, :FB]], axis=1)
    out = jax.nn.sigmoid(agg / (s + 1e-16))
    sg = sg_ref[...]
    return sg * out + (1.0 - sg) * x_ref[...]


def _combine_prep_block(p_ref, sg_ref, x_ref, wl_ref, av_ref, w_ref, b_ref,
                        x1_ref, hc_ref, ad_ref, as_ref, sg1_ref):
    x1 = _gate_combine(p_ref, sg_ref, x_ref)
    x1_ref[...] = x1
    h = jnp.dot(x1, wl_ref[...], preferred_element_type=jnp.float32)
    hc_ref[...] = _split_tables(h)
    coef = lax.dot_general(h, av_ref[...], (((1,), (1,)), ((), ())),
                           preferred_element_type=jnp.float32)
    ad_ref[...] = coef[:, 0].reshape(1, 1, BR)
    as_ref[...] = coef[:, 1].reshape(1, 1, BR)
    sg1_ref[...] = jax.nn.sigmoid(
        jnp.dot(x1, w_ref[...], preferred_element_type=jnp.float32) + b_ref[...])


def _tc_combine_prep(parts, sg, x, wl, av, w, b):
    return pl.pallas_call(
        _combine_prep_block,
        grid=(GRID,),
        in_specs=[
            pl.BlockSpec((2, BR, DPA), lambda i: (0, i, 0)),
            pl.BlockSpec((BR, D), lambda i: (i, 0)),
            pl.BlockSpec((BR, D), lambda i: (i, 0)),
            pl.BlockSpec((D, D), lambda i: (0, 0)),
            pl.BlockSpec((2, D), lambda i: (0, 0)),
            pl.BlockSpec((D, D), lambda i: (0, 0)),
            pl.BlockSpec((1, D), lambda i: (0, 0)),
        ],
        out_specs=[
            pl.BlockSpec((BR, D), lambda i: (i, 0)),
            pl.BlockSpec((2, BR, DPA), lambda i: (0, i, 0)),
            pl.BlockSpec((1, 1, BR), lambda i: (i, 0, 0)),
            pl.BlockSpec((1, 1, BR), lambda i: (i, 0, 0)),
            pl.BlockSpec((BR, D), lambda i: (i, 0)),
        ],
        out_shape=[
            jax.ShapeDtypeStruct((N, D), jnp.float32),
            jax.ShapeDtypeStruct((2, N, DPA), jnp.float32),
            jax.ShapeDtypeStruct((GRID, 1, BR), jnp.float32),
            jax.ShapeDtypeStruct((GRID, 1, BR), jnp.float32),
            jax.ShapeDtypeStruct((N, D), jnp.float32),
        ],
    )(parts, sg, x, wl, av, w, b)


def _combine_final_block(p_ref, sg_ref, x_ref, x2p_ref):
    x2 = _gate_combine(p_ref, sg_ref, x_ref)
    x2p_ref[...] = jnp.concatenate(
        [x2, jnp.zeros((BR, DP - D), jnp.float32)], axis=1)


def _tc_combine_final(parts, sg, x):
    return pl.pallas_call(
        _combine_final_block,
        grid=(GRID,),
        in_specs=[
            pl.BlockSpec((2, BR, DPA), lambda i: (0, i, 0)),
            pl.BlockSpec((BR, D), lambda i: (i, 0)),
            pl.BlockSpec((BR, D), lambda i: (i, 0)),
        ],
        out_specs=[pl.BlockSpec((BR, DP), lambda i: (i, 0))],
        out_shape=[jax.ShapeDtypeStruct((N, DP), jnp.float32)],
    )(parts, sg, x)


def _sc_edges(src, dst, adst, asrc, hcat, zeros):
    mesh = plsc.VectorSubcoreMesh(core_axis_name="c", subcore_axis_name="s")
    def nbuf_scratch():
        return [
            pltpu.VMEM((CH,), jnp.int32),
            pltpu.VMEM((CH,), jnp.int32),
            pltpu.VMEM((CH,), jnp.int32),
            pltpu.VMEM((CH,), jnp.int32),
            pltpu.VMEM((CH,), jnp.float32),
            pltpu.VMEM((CH, DPA), jnp.float32),
            pltpu.VMEM((CH, DPA), jnp.float32),
            pltpu.SemaphoreType.DMA,
            pltpu.SemaphoreType.DMA,
        ]

    @functools.partial(
        pl.kernel,
        mesh=mesh,
        out_type=jax.ShapeDtypeStruct((2, N, DPA), jnp.float32),
        compiler_params=_SC_PARAMS,
        scratch_types=[
            pltpu.VMEM((N,), jnp.float32),
            pltpu.VMEM((N,), jnp.float32),
        ] + nbuf_scratch() + nbuf_scratch() + [
            pltpu.SemaphoreType.DMA,
            pltpu.VMEM_SHARED((N, DPA), jnp.float32),
        ],
    )
    def body(src_h, dst_h, adst_h, asrc_h, hcat_h, zeros_h, parts_h,
             adst_v, asrc_v, *bufs_and_acc):
        bufs = bufs_and_acc[:18]
        ssem = bufs_and_acc[18]
        acc = bufs_and_acc[19]
        cid = lax.axis_index("c")
        sid = lax.axis_index("s")
        pltpu.sync_copy(adst_h, adst_v)
        pltpu.sync_copy(asrc_h, asrc_v)
        pltpu.sync_copy(zeros_h, acc.at[pl.ds(sid * ROWS_PT, ROWS_PT)])
        plsc.subcore_barrier()
        rowoff = cid * N
        b0 = bufs[:9]
        b1 = bufs[9:]

        def chunk_off(c):
            return pl.multiple_of(
                jnp.minimum(sid * EPS + c * CH, E - CH), 8)

        def p1(c, buf):
            srcv, dstv, _, _, _, _, _, isem, _ = buf
            off = chunk_off(c)
            pltpu.async_copy(src_h.at[pl.ds(off, CH)], srcv, isem)
            pltpu.async_copy(dst_h.at[pl.ds(off, CH)], dstv, isem)

        def p1_wait(c, buf):
            srcv, dstv, _, _, _, _, _, isem, _ = buf
            off = chunk_off(c)
            pltpu.make_async_copy(src_h.at[pl.ds(off, CH)], srcv, isem).wait()
            pltpu.make_async_copy(dst_h.at[pl.ds(off, CH)], dstv, isem).wait()

        def p2(c, buf):
            srcv, dstv, sdstv, gidx, exv, rows, srows, isem, gsem = buf
            p1_wait(c, buf)
            for j in range(CH // 16):
                sl = pl.ds(j * 16, 16)
                gidx[sl] = srcv[sl] + rowoff
            pltpu.async_copy(hcat_h.at[gidx], rows, gsem)
            for j in range(CH // 16):
                sl = pl.ds(j * 16, 16)
                sdstv[sl] = dstv[sl]
                e = (plsc.load_gather(adst_v, [dstv[sl]])
                     + plsc.load_gather(asrc_v, [srcv[sl]]))
                e = jnp.where(e > 0, e, ALPHA * e)
                exv[sl] = jnp.exp(e)

        def drain_scatter(buf):
            srows = buf[6]
            pltpu.make_async_copy(hcat_h.at[pl.ds(0, CH)], srows, ssem).wait()

        def p3(buf, prev_buf):
            srcv, dstv, sdstv, gidx, exv, rows, srows, isem, gsem = buf
            pltpu.make_async_copy(hcat_h.at[gidx], rows, gsem).wait()
            for j in range(CH // 16):
                ev = exv[pl.ds(j * 16, 16)]
                for l in range(16):
                    w = lax.reshape(lax.slice(ev, (l,), (l + 1,)), ())
                    r = j * 16 + l
                    for k in range(DPA // 16):
                        sl = pl.ds(k * 16, 16)
                        srows[r, sl] = rows[r, sl] * w
            drain_scatter(prev_buf)
            pltpu.async_copy(srows, acc.at[sdstv], ssem, add=True)

        for j in range(CH // 16):
            b1[2][pl.ds(j * 16, 16)] = jnp.zeros((16,), jnp.int32)
        pltpu.sync_copy(zeros_h.at[pl.ds(0, CH)], b1[6])
        pltpu.async_copy(b1[6], acc.at[b1[2]], ssem, add=True)
        p1(0, b0)
        p1(1, b1)
        p2(0, b0)
        p2(1, b1)
        p1(2, b0)
        p1(3, b1)

        def superstep(g, carry):
            c0 = 2 * g
            p3(b0, b1)
            p3(b1, b0)
            p2(c0 + 2, b0)
            p2(c0 + 3, b1)
            p1(c0 + 4, b0)
            p1(c0 + 5, b1)
            return carry

        lax.fori_loop(0, NCH // 2, superstep, 0)

        for c, buf in ((NCH + 2, b0), (NCH + 3, b1)):
            srcv, dstv, sdstv, gidx, exv, rows, srows, isem, gsem = buf
            p1_wait(c, buf)
            pltpu.make_async_copy(hcat_h.at[gidx], rows, gsem).wait()
        drain_scatter(b1)
        plsc.subcore_barrier()
        pltpu.sync_copy(acc.at[pl.ds(sid * ROWS_PT, ROWS_PT)],
                        parts_h.at[cid, pl.ds(sid * ROWS_PT, ROWS_PT)])

    return body(src, dst, adst, asrc, hcat, zeros)


def _sc_batch_gather(bh, bt, br, x2p, relp):
    mesh = plsc.VectorSubcoreMesh(core_axis_name="c", subcore_axis_name="s")
    bpw = B // 32

    @functools.partial(
        pl.kernel,
        mesh=mesh,
        out_type=(
            jax.ShapeDtypeStruct((B, DP), jnp.float32),
            jax.ShapeDtypeStruct((B, DP), jnp.float32),
            jax.ShapeDtypeStruct((B, DP), jnp.float32),
        ),
        compiler_params=_SC_PARAMS,
        scratch_types=[
            pltpu.VMEM((bpw,), jnp.int32),
            pltpu.VMEM((bpw, DP), jnp.float32),
            pltpu.SemaphoreType.DMA,
        ],
    )
    def body(bh_h, bt_h, br_h, x2p_h, relp_h, oh_h, ot_h, orr_h,
             idxv, rows, sem):
        cid = lax.axis_index("c")
        sid = lax.axis_index("s")
        base = pl.multiple_of((cid * 16 + sid) * bpw, 8)
        for idx_h, tab_h, out_h in ((bh_h, x2p_h, oh_h),
                                    (bt_h, x2p_h, ot_h),
                                    (br_h, relp_h, orr_h)):
            pltpu.sync_copy(idx_h.at[pl.ds(base, bpw)], idxv)
            pltpu.async_copy(tab_h.at[idxv], rows, sem).wait()
            pltpu.sync_copy(rows, out_h.at[pl.ds(base, bpw)])

    return body(bh, bt, br, x2p, relp)


def kernel(batch_h, batch_r, batch_t, edge_index, ent_embed, rel_embed,
           W, b, Wl0, a0, Wl1, a1):
    src = edge_index[0].astype(jnp.int32)
    dst = edge_index[1].astype(jnp.int32)
    av0 = a0.reshape(2, D)
    av1 = a1.reshape(2, D)
    zeros = jnp.zeros((ROWS_PT, DPA), jnp.float32)

    hc0, ad0, as0, sg0 = _tc_prep(ent_embed, Wl0, av0, W, b)
    parts0 = _sc_edges(src, dst, ad0.reshape(N), as0.reshape(N),
                       hc0.reshape(2 * N, DPA), zeros)
    x1, hc1, ad1, as1, sg1 = _tc_combine_prep(parts0, sg0, ent_embed,
                                              Wl1, av1, W, b)
    parts1 = _sc_edges(src, dst, ad1.reshape(N), as1.reshape(N),
                       hc1.reshape(2 * N, DPA), zeros)
    (x2p,) = _tc_combine_final(parts1, sg1, x1)

    relp = jnp.pad(rel_embed, ((0, 0), (0, DP - D)))
    oh, ot, orr = _sc_batch_gather(batch_h.astype(jnp.int32),
                                   batch_t.astype(jnp.int32),
                                   batch_r.astype(jnp.int32), x2p, relp)
    return oh[:, :D], orr[:, :D], ot[:, :D]
